# Initial kernel scaffold; baseline (speedup 1.0000x reference)
import jax
import jax.numpy as jnp
from jax.experimental import pallas as pl


def kernel(attention, context, K):
    raise NotImplementedError("write your pallas kernel here")


# speedup vs baseline: 3.3189x; 3.3189x over previous
import jax
import jax.numpy as jnp
from jax import lax
from jax.experimental import pallas as pl
from jax.experimental.pallas import tpu as pltpu
from jax.experimental.pallas import tpu_sc as plsc

B = 64
S = 4096
D = 1024
KV = 32

NLANE = 16
NSEG = 16
SEGLEN = S // NSEG
NCHUNK = SEGLEN // NLANE

NC = 2
NS = 16
NW = NC * NS
ROWS_PER_W = B // NW

_INT_MAX = 2**31 - 1
_NEG_INF = float("-inf")


def _dyn_gather(x, idx):
    return lax.gather(
        x, idx[:, None],
        dimension_numbers=lax.GatherDimensionNumbers(
            offset_dims=(), collapsed_slice_dims=(0,), start_index_map=(0,)),
        slice_sizes=(1,),
        mode=lax.GatherScatterMode.PROMISE_IN_BOUNDS,
    )


def _tec_body(att_hbm, ctx_hbm, out_hbm, att_v, idx_v, rows_v, sem):
    wid = lax.axis_index("s") * NC + lax.axis_index("c")
    lanes = lax.iota(jnp.int32, NLANE)

    def allmax(v):
        for sh in (8, 4, 2, 1):
            v = jnp.maximum(v, _dyn_gather(v, jnp.bitwise_xor(lanes, sh)))
        return v

    def allmin(v):
        for sh in (8, 4, 2, 1):
            v = jnp.minimum(v, _dyn_gather(v, jnp.bitwise_xor(lanes, sh)))
        return v

    def seg_scan(seg_base):
        def chunk(j, carry):
            m, g = carry
            base = seg_base + j * NLANE
            v = att_v[pl.ds(base, NLANE)]
            upd = v > m
            return (jnp.where(upd, v, m), jnp.where(upd, base + lanes, g))

        m0 = jnp.full((NLANE,), _NEG_INF, jnp.float32)
        g0 = jnp.zeros((NLANE,), jnp.int32)
        m, g = lax.fori_loop(0, NCHUNK, chunk, (m0, g0))
        smax = allmax(m)
        sarg = allmin(jnp.where(m == smax, g, _INT_MAX))
        return smax, sarg

    for r in range(ROWS_PER_W):
        b = wid * ROWS_PER_W + r
        pltpu.sync_copy(att_hbm.at[pl.ds(b * S, S)], att_v)

        def build(s, carry):
            segmax, segarg = carry
            smax, sarg = seg_scan(s * SEGLEN)
            lane_is_s = lanes == s
            return (jnp.where(lane_is_s, smax, segmax),
                    jnp.where(lane_is_s, sarg, segarg))

        segmax, segarg = lax.fori_loop(
            0, NSEG, build,
            (jnp.full((NLANE,), _NEG_INF, jnp.float32),
             jnp.zeros((NLANE,), jnp.int32)))

        def extract(k, carry):
            segmax, segarg, idx_lo, idx_hi = carry
            gmax = allmax(segmax)
            g_vec = allmin(jnp.where(segmax == gmax, segarg, _INT_MAX))
            g = g_vec[0]
            idx_lo = jnp.where(lanes == k, b * S + g_vec, idx_lo)
            idx_hi = jnp.where(lanes == (k - NLANE), b * S + g_vec, idx_hi)
            lane = jnp.bitwise_and(g, NLANE - 1)
            cbase = g - lane
            v = att_v[pl.ds(cbase, NLANE)]
            att_v[pl.ds(cbase, NLANE)] = jnp.where(lanes == lane, _NEG_INF, v)
            s_star = lax.shift_right_logical(g, 8)
            smax, sarg = seg_scan(s_star * SEGLEN)
            lane_is_s = lanes == s_star
            return (jnp.where(lane_is_s, smax, segmax),
                    jnp.where(lane_is_s, sarg, segarg),
                    idx_lo, idx_hi)

        z = jnp.zeros((NLANE,), jnp.int32)
        _, _, idx_lo, idx_hi = lax.fori_loop(
            0, KV, extract, (segmax, segarg, z, z))
        idx_v[pl.ds(0, NLANE)] = idx_lo
        idx_v[pl.ds(NLANE, NLANE)] = idx_hi

        pltpu.async_copy(ctx_hbm.at[idx_v], rows_v, sem).wait()
        pltpu.sync_copy(rows_v, out_hbm.at[pl.ds(b * KV, KV)])


_sc_call = pl.kernel(
    _tec_body,
    out_type=jax.ShapeDtypeStruct((B * KV, D), jnp.float32),
    mesh=plsc.VectorSubcoreMesh(core_axis_name="c", subcore_axis_name="s"),
    scratch_types=[
        pltpu.VMEM((S,), jnp.float32),
        pltpu.VMEM((KV,), jnp.int32),
        pltpu.VMEM((KV, D), jnp.float32),
        pltpu.SemaphoreType.DMA,
    ],
)


def kernel(attention, context, K):
    del K
    out = _sc_call(attention.reshape(B * S), context.reshape(B * S, D))
    return out.reshape(B, KV, D)


# speedup vs baseline: 3.5994x; 1.0845x over previous
import jax
import jax.numpy as jnp
from jax import lax
from jax.experimental import pallas as pl
from jax.experimental.pallas import tpu as pltpu
from jax.experimental.pallas import tpu_sc as plsc

B = 64
S = 4096
D = 1024
KV = 32

NLANE = 16
NSEG = 16
SEGLEN = S // NSEG
NCHUNK = SEGLEN // NLANE

NC = 2
NS = 16
NW = NC * NS
ROWS_PER_W = B // NW

_INT_MAX = 2**31 - 1
_NEG_INF = float("-inf")


def _dyn_gather(x, idx):
    return lax.gather(
        x, idx[:, None],
        dimension_numbers=lax.GatherDimensionNumbers(
            offset_dims=(), collapsed_slice_dims=(0,), start_index_map=(0,)),
        slice_sizes=(1,),
        mode=lax.GatherScatterMode.PROMISE_IN_BOUNDS,
    )


def _tec_body(att_hbm, ctx_hbm, out_hbm, att0_v, att1_v, idx0_v, idx1_v,
              rows0_v, rows1_v, s_a0, s_a1, s_g0, s_g1, s_o0, s_o1):
    wid = lax.axis_index("s") * NC + lax.axis_index("c")
    lanes = lax.iota(jnp.int32, NLANE)

    def allmax(v):
        for sh in (8, 4, 2, 1):
            v = jnp.maximum(v, _dyn_gather(v, jnp.bitwise_xor(lanes, sh)))
        return v

    def allmin(v):
        for sh in (8, 4, 2, 1):
            v = jnp.minimum(v, _dyn_gather(v, jnp.bitwise_xor(lanes, sh)))
        return v

    def topk_row(att_v, idx_v, b):
        def seg_scan(seg_base):
            def chunk(j, carry):
                m, g = carry
                base = seg_base + j * NLANE
                v = att_v[pl.ds(base, NLANE)]
                upd = v > m
                return (jnp.where(upd, v, m), jnp.where(upd, base + lanes, g))

            m0 = jnp.full((NLANE,), _NEG_INF, jnp.float32)
            g0 = jnp.zeros((NLANE,), jnp.int32)
            m, g = lax.fori_loop(0, NCHUNK, chunk, (m0, g0))
            smax = allmax(m)
            sarg = allmin(jnp.where(m == smax, g, _INT_MAX))
            return smax, sarg

        def build(s, carry):
            segmax, segarg = carry
            smax, sarg = seg_scan(s * SEGLEN)
            lane_is_s = lanes == s
            return (jnp.where(lane_is_s, smax, segmax),
                    jnp.where(lane_is_s, sarg, segarg))

        segmax, segarg = lax.fori_loop(
            0, NSEG, build,
            (jnp.full((NLANE,), _NEG_INF, jnp.float32),
             jnp.zeros((NLANE,), jnp.int32)))

        def extract(k, carry):
            segmax, segarg, idx_lo, idx_hi = carry
            gmax = allmax(segmax)
            g_vec = allmin(jnp.where(segmax == gmax, segarg, _INT_MAX))
            g = g_vec[0]
            idx_lo = jnp.where(lanes == k, b * S + g_vec, idx_lo)
            idx_hi = jnp.where(lanes == (k - NLANE), b * S + g_vec, idx_hi)
            lane = jnp.bitwise_and(g, NLANE - 1)
            cbase = g - lane
            v = att_v[pl.ds(cbase, NLANE)]
            att_v[pl.ds(cbase, NLANE)] = jnp.where(lanes == lane, _NEG_INF, v)
            s_star = lax.shift_right_logical(g, 8)
            smax, sarg = seg_scan(s_star * SEGLEN)
            lane_is_s = lanes == s_star
            return (jnp.where(lane_is_s, smax, segmax),
                    jnp.where(lane_is_s, sarg, segarg),
                    idx_lo, idx_hi)

        z = jnp.zeros((NLANE,), jnp.int32)
        _, _, idx_lo, idx_hi = lax.fori_loop(
            0, KV, extract, (segmax, segarg, z, z))
        idx_v[pl.ds(0, NLANE)] = idx_lo
        idx_v[pl.ds(NLANE, NLANE)] = idx_hi

    b0 = wid * ROWS_PER_W
    b1 = b0 + 1
    a0 = pltpu.async_copy(att_hbm.at[b0], att0_v, s_a0)
    a1 = pltpu.async_copy(att_hbm.at[b1], att1_v, s_a1)
    a0.wait()
    topk_row(att0_v, idx0_v, b0)
    g0 = pltpu.async_copy(ctx_hbm.at[idx0_v], rows0_v, s_g0)
    a1.wait()
    topk_row(att1_v, idx1_v, b1)
    g1 = pltpu.async_copy(ctx_hbm.at[idx1_v], rows1_v, s_g1)
    g0.wait()
    o0 = pltpu.async_copy(rows0_v, out_hbm.at[pl.ds(b0 * KV, KV)], s_o0)
    g1.wait()
    o1 = pltpu.async_copy(rows1_v, out_hbm.at[pl.ds(b1 * KV, KV)], s_o1)
    o0.wait()
    o1.wait()


_sc_call = pl.kernel(
    _tec_body,
    out_type=jax.ShapeDtypeStruct((B * KV, D), jnp.float32),
    mesh=plsc.VectorSubcoreMesh(core_axis_name="c", subcore_axis_name="s"),
    scratch_types=[
        pltpu.VMEM((S,), jnp.float32),
        pltpu.VMEM((S,), jnp.float32),
        pltpu.VMEM((KV,), jnp.int32),
        pltpu.VMEM((KV,), jnp.int32),
        pltpu.VMEM((KV, D), jnp.float32),
        pltpu.VMEM((KV, D), jnp.float32),
        pltpu.SemaphoreType.DMA,
        pltpu.SemaphoreType.DMA,
        pltpu.SemaphoreType.DMA,
        pltpu.SemaphoreType.DMA,
        pltpu.SemaphoreType.DMA,
        pltpu.SemaphoreType.DMA,
    ],
)


def kernel(attention, context, K):
    del K
    out = _sc_call(attention, context.reshape(B * S, D))
    return out.reshape(B, KV, D)


# speedup vs baseline: 3.9895x; 1.1084x over previous
import jax
import jax.numpy as jnp
from jax import lax
from jax.experimental import pallas as pl
from jax.experimental.pallas import tpu as pltpu
from jax.experimental.pallas import tpu_sc as plsc

B = 64
S = 4096
D = 1024
KV = 32

NLANE = 16
NSEG = 16
SEGLEN = S // NSEG
NCHUNK = SEGLEN // NLANE

NC = 2
NS = 16
NW = NC * NS
ROWS_PER_W = B // NW

_INT_MAX = 2**31 - 1
_NEG_INF = float("-inf")


def _dyn_gather(x, idx):
    return lax.gather(
        x, idx[:, None],
        dimension_numbers=lax.GatherDimensionNumbers(
            offset_dims=(), collapsed_slice_dims=(0,), start_index_map=(0,)),
        slice_sizes=(1,),
        mode=lax.GatherScatterMode.PROMISE_IN_BOUNDS,
    )


def _tec_body(att_hbm, ctx_hbm, out_hbm, att0_v, att1_v, idx0_v, idx1_v,
              rows0_v, rows1_v, s_a0, s_a1, s_g0, s_g1, s_o0, s_o1):
    wid = lax.axis_index("s") * NC + lax.axis_index("c")
    lanes = lax.iota(jnp.int32, NLANE)

    def allmax(v):
        for sh in (8, 4, 2, 1):
            v = jnp.maximum(v, _dyn_gather(v, jnp.bitwise_xor(lanes, sh)))
        return v

    def allmin(v):
        for sh in (8, 4, 2, 1):
            v = jnp.minimum(v, _dyn_gather(v, jnp.bitwise_xor(lanes, sh)))
        return v

    def topk_row(att_v, idx_v, b):
        def seg_scan(seg_base):
            m = att_v[pl.ds(seg_base, NLANE)]
            g = seg_base + lanes
            for j in range(1, NCHUNK):
                base = seg_base + j * NLANE
                v = att_v[pl.ds(base, NLANE)]
                upd = v > m
                m = jnp.where(upd, v, m)
                g = jnp.where(upd, base + lanes, g)
            smax = allmax(m)
            sarg = allmin(jnp.where(m == smax, g, _INT_MAX))
            return smax, sarg

        def build(s, carry):
            segmax, segarg = carry
            smax, sarg = seg_scan(s * SEGLEN)
            lane_is_s = lanes == s
            return (jnp.where(lane_is_s, smax, segmax),
                    jnp.where(lane_is_s, sarg, segarg))

        segmax, segarg = lax.fori_loop(
            0, NSEG, build,
            (jnp.full((NLANE,), _NEG_INF, jnp.float32),
             jnp.zeros((NLANE,), jnp.int32)))

        def extract(k, carry):
            segmax, segarg, idx_lo, idx_hi = carry
            gmax = allmax(segmax)
            g_vec = allmin(jnp.where(segmax == gmax, segarg, _INT_MAX))
            g = g_vec[0]
            idx_lo = jnp.where(lanes == k, b * S + g_vec, idx_lo)
            idx_hi = jnp.where(lanes == (k - NLANE), b * S + g_vec, idx_hi)
            lane = jnp.bitwise_and(g, NLANE - 1)
            cbase = g - lane
            v = att_v[pl.ds(cbase, NLANE)]
            att_v[pl.ds(cbase, NLANE)] = jnp.where(lanes == lane, _NEG_INF, v)
            s_star = lax.shift_right_logical(g, 8)
            smax, sarg = seg_scan(s_star * SEGLEN)
            lane_is_s = lanes == s_star
            return (jnp.where(lane_is_s, smax, segmax),
                    jnp.where(lane_is_s, sarg, segarg),
                    idx_lo, idx_hi)

        z = jnp.zeros((NLANE,), jnp.int32)
        _, _, idx_lo, idx_hi = lax.fori_loop(
            0, KV, extract, (segmax, segarg, z, z))
        idx_v[pl.ds(0, NLANE)] = idx_lo
        idx_v[pl.ds(NLANE, NLANE)] = idx_hi

    b0 = wid * ROWS_PER_W
    b1 = b0 + 1
    a0 = pltpu.async_copy(att_hbm.at[b0], att0_v, s_a0)
    a1 = pltpu.async_copy(att_hbm.at[b1], att1_v, s_a1)
    a0.wait()
    topk_row(att0_v, idx0_v, b0)
    g0 = pltpu.async_copy(ctx_hbm.at[idx0_v], rows0_v, s_g0)
    a1.wait()
    topk_row(att1_v, idx1_v, b1)
    g1 = pltpu.async_copy(ctx_hbm.at[idx1_v], rows1_v, s_g1)
    g0.wait()
    o0 = pltpu.async_copy(rows0_v, out_hbm.at[pl.ds(b0 * KV, KV)], s_o0)
    g1.wait()
    o1 = pltpu.async_copy(rows1_v, out_hbm.at[pl.ds(b1 * KV, KV)], s_o1)
    o0.wait()
    o1.wait()


_sc_call = pl.kernel(
    _tec_body,
    out_type=jax.ShapeDtypeStruct((B * KV, D), jnp.float32),
    mesh=plsc.VectorSubcoreMesh(core_axis_name="c", subcore_axis_name="s"),
    scratch_types=[
        pltpu.VMEM((S,), jnp.float32),
        pltpu.VMEM((S,), jnp.float32),
        pltpu.VMEM((KV,), jnp.int32),
        pltpu.VMEM((KV,), jnp.int32),
        pltpu.VMEM((KV, D), jnp.float32),
        pltpu.VMEM((KV, D), jnp.float32),
        pltpu.SemaphoreType.DMA,
        pltpu.SemaphoreType.DMA,
        pltpu.SemaphoreType.DMA,
        pltpu.SemaphoreType.DMA,
        pltpu.SemaphoreType.DMA,
        pltpu.SemaphoreType.DMA,
    ],
)


def kernel(attention, context, K):
    del K
    out = _sc_call(attention, context.reshape(B * S, D))
    return out.reshape(B, KV, D)


# speedup vs baseline: 4.2169x; 1.0570x over previous
import jax
import jax.numpy as jnp
from jax import lax
from jax.experimental import pallas as pl
from jax.experimental.pallas import tpu as pltpu
from jax.experimental.pallas import tpu_sc as plsc

B = 64
S = 4096
D = 1024
KV = 32

NLANE = 16
NSEG = 16
SEGLEN = S // NSEG
NCHUNK = SEGLEN // NLANE

NC = 2
NS = 16
NW = NC * NS
ROWS_PER_W = B // NW

_INT_MAX = 2**31 - 1
_NEG_INF = float("-inf")


def _dyn_gather(x, idx):
    return lax.gather(
        x, idx[:, None],
        dimension_numbers=lax.GatherDimensionNumbers(
            offset_dims=(), collapsed_slice_dims=(0,), start_index_map=(0,)),
        slice_sizes=(1,),
        mode=lax.GatherScatterMode.PROMISE_IN_BOUNDS,
    )


def _tec_body(att_hbm, ctx_hbm, out_hbm, att0_v, att1_v, rows0_v, rows1_v,
              s_a0, s_a1, s_g0, s_g1, s_o0, s_o1):
    wid = lax.axis_index("s") * NC + lax.axis_index("c")
    lanes = lax.iota(jnp.int32, NLANE)

    def allmax(v):
        for sh in (8, 4, 2, 1):
            v = jnp.maximum(v, _dyn_gather(v, jnp.bitwise_xor(lanes, sh)))
        return v

    def allmin(v):
        for sh in (8, 4, 2, 1):
            v = jnp.minimum(v, _dyn_gather(v, jnp.bitwise_xor(lanes, sh)))
        return v

    def seg_scan(att_v, seg_base):
        m = att_v[pl.ds(seg_base, NLANE)]
        g = seg_base + lanes
        for j in range(1, NCHUNK):
            base = seg_base + j * NLANE
            v = att_v[pl.ds(base, NLANE)]
            upd = v > m
            m = jnp.where(upd, v, m)
            g = jnp.where(upd, base + lanes, g)
        smax = allmax(m)
        sarg = allmin(jnp.where(m == smax, g, _INT_MAX))
        return smax, sarg

    def build_one(att_v, s, segmax, segarg):
        smax, sarg = seg_scan(att_v, s * SEGLEN)
        lane_is_s = lanes == s
        return (jnp.where(lane_is_s, smax, segmax),
                jnp.where(lane_is_s, sarg, segarg))

    def extract_one(att_v, segmax, segarg):
        gmax = allmax(segmax)
        g_vec = allmin(jnp.where(segmax == gmax, segarg, _INT_MAX))
        g = g_vec[0]
        lane = jnp.bitwise_and(g, NLANE - 1)
        cbase = g - lane
        v = att_v[pl.ds(cbase, NLANE)]
        att_v[pl.ds(cbase, NLANE)] = jnp.where(lanes == lane, _NEG_INF, v)
        s_star = lax.shift_right_logical(g, 8)
        smax, sarg = seg_scan(att_v, s_star * SEGLEN)
        lane_is_s = lanes == s_star
        return (g_vec,
                jnp.where(lane_is_s, smax, segmax),
                jnp.where(lane_is_s, sarg, segarg))

    b0 = wid * ROWS_PER_W
    b1 = b0 + 1

    a0 = pltpu.async_copy(att_hbm.at[b0], att0_v, s_a0)
    a1 = pltpu.async_copy(att_hbm.at[b1], att1_v, s_a1)
    a0.wait()
    a1.wait()

    def build(s, carry):
        sm0, sa0, sm1, sa1 = carry
        sm0, sa0 = build_one(att0_v, s, sm0, sa0)
        sm1, sa1 = build_one(att1_v, s, sm1, sa1)
        return (sm0, sa0, sm1, sa1)

    neg = jnp.full((NLANE,), _NEG_INF, jnp.float32)
    zer = jnp.zeros((NLANE,), jnp.int32)
    sm0, sa0, sm1, sa1 = lax.fori_loop(0, NSEG, build, (neg, zer, neg, zer))

    def extract_lo(k, carry):
        sm0, sa0, sm1, sa1, lo0, lo1 = carry
        g0_vec, sm0, sa0 = extract_one(att0_v, sm0, sa0)
        g1_vec, sm1, sa1 = extract_one(att1_v, sm1, sa1)
        lane_is_k = lanes == k
        lo0 = jnp.where(lane_is_k, b0 * S + g0_vec, lo0)
        lo1 = jnp.where(lane_is_k, b1 * S + g1_vec, lo1)
        return (sm0, sa0, sm1, sa1, lo0, lo1)

    sm0, sa0, sm1, sa1, lo0, lo1 = lax.fori_loop(
        0, NLANE, extract_lo, (sm0, sa0, sm1, sa1, zer, zer))

    g0a = pltpu.async_copy(ctx_hbm.at[lo0], rows0_v.at[pl.ds(0, NLANE)], s_g0)
    g1a = pltpu.async_copy(ctx_hbm.at[lo1], rows1_v.at[pl.ds(0, NLANE)], s_g1)

    def extract_hi(k, carry):
        sm0, sa0, sm1, sa1, hi0, hi1 = carry
        g0_vec, sm0, sa0 = extract_one(att0_v, sm0, sa0)
        g1_vec, sm1, sa1 = extract_one(att1_v, sm1, sa1)
        lane_is_k = lanes == k
        hi0 = jnp.where(lane_is_k, b0 * S + g0_vec, hi0)
        hi1 = jnp.where(lane_is_k, b1 * S + g1_vec, hi1)
        return (sm0, sa0, sm1, sa1, hi0, hi1)

    _, _, _, _, hi0, hi1 = lax.fori_loop(
        0, NLANE, extract_hi, (sm0, sa0, sm1, sa1, zer, zer))

    g0b = pltpu.async_copy(ctx_hbm.at[hi0], rows0_v.at[pl.ds(NLANE, NLANE)],
                           s_g0)
    g1b = pltpu.async_copy(ctx_hbm.at[hi1], rows1_v.at[pl.ds(NLANE, NLANE)],
                           s_g1)

    g0a.wait()
    g0b.wait()
    o0 = pltpu.async_copy(rows0_v, out_hbm.at[pl.ds(b0 * KV, KV)], s_o0)
    g1a.wait()
    g1b.wait()
    o1 = pltpu.async_copy(rows1_v, out_hbm.at[pl.ds(b1 * KV, KV)], s_o1)
    o0.wait()
    o1.wait()


_sc_call = pl.kernel(
    _tec_body,
    out_type=jax.ShapeDtypeStruct((B * KV, D), jnp.float32),
    mesh=plsc.VectorSubcoreMesh(core_axis_name="c", subcore_axis_name="s"),
    scratch_types=[
        pltpu.VMEM((S,), jnp.float32),
        pltpu.VMEM((S,), jnp.float32),
        pltpu.VMEM((KV, D), jnp.float32),
        pltpu.VMEM((KV, D), jnp.float32),
        pltpu.SemaphoreType.DMA,
        pltpu.SemaphoreType.DMA,
        pltpu.SemaphoreType.DMA,
        pltpu.SemaphoreType.DMA,
        pltpu.SemaphoreType.DMA,
        pltpu.SemaphoreType.DMA,
    ],
)


def kernel(attention, context, K):
    del K
    out = _sc_call(attention, context.reshape(B * S, D))
    return out.reshape(B, KV, D)


# speedup vs baseline: 4.2192x; 1.0005x over previous
import jax
import jax.numpy as jnp
from jax import lax
from jax.experimental import pallas as pl
from jax.experimental.pallas import tpu as pltpu
from jax.experimental.pallas import tpu_sc as plsc

B = 64
S = 4096
D = 1024
KV = 32

NLANE = 16
NSEG = 16
SEGLEN = S // NSEG
NCHUNK = SEGLEN // NLANE

NC = 2
NS = 16
NW = NC * NS
ROWS_PER_W = B // NW

_INT_MAX = 2**31 - 1
_NEG_INF = float("-inf")


def _dyn_gather(x, idx):
    return lax.gather(
        x, idx[:, None],
        dimension_numbers=lax.GatherDimensionNumbers(
            offset_dims=(), collapsed_slice_dims=(0,), start_index_map=(0,)),
        slice_sizes=(1,),
        mode=lax.GatherScatterMode.PROMISE_IN_BOUNDS,
    )


def _tec_body(att_hbm, ctx_hbm, out_hbm, att0_v, att1_v, rows0_v, rows1_v,
              s_a0, s_a1, s_g0, s_g1, s_o0, s_o1):
    wid = lax.axis_index("s") * NC + lax.axis_index("c")
    lanes = lax.iota(jnp.int32, NLANE)

    def allmax(v):
        for sh in (8, 4, 2, 1):
            v = jnp.maximum(v, _dyn_gather(v, jnp.bitwise_xor(lanes, sh)))
        return v

    def allmin(v):
        for sh in (8, 4, 2, 1):
            v = jnp.minimum(v, _dyn_gather(v, jnp.bitwise_xor(lanes, sh)))
        return v

    def seg_scan(att_v, seg_base):
        m = att_v[pl.ds(seg_base, NLANE)]
        g = seg_base + lanes
        for j in range(1, NCHUNK):
            base = seg_base + j * NLANE
            v = att_v[pl.ds(base, NLANE)]
            upd = v > m
            m = jnp.where(upd, v, m)
            g = jnp.where(upd, base + lanes, g)
        smax = allmax(m)
        sarg = allmin(jnp.where(m == smax, g, _INT_MAX))
        return smax, sarg

    def build_one(att_v, s, segmax, segarg):
        smax, sarg = seg_scan(att_v, s * SEGLEN)
        lane_is_s = lanes == s
        return (jnp.where(lane_is_s, smax, segmax),
                jnp.where(lane_is_s, sarg, segarg))

    def extract_one(att_v, segmax, segarg):
        gmax = allmax(segmax)
        g_vec = allmin(jnp.where(segmax == gmax, segarg, _INT_MAX))
        g = g_vec[0]
        lane = jnp.bitwise_and(g, NLANE - 1)
        cbase = g - lane
        v = att_v[pl.ds(cbase, NLANE)]
        att_v[pl.ds(cbase, NLANE)] = jnp.where(lanes == lane, _NEG_INF, v)
        s_star = lax.shift_right_logical(g, 8)
        smax, sarg = seg_scan(att_v, s_star * SEGLEN)
        lane_is_s = lanes == s_star
        return (g_vec,
                jnp.where(lane_is_s, smax, segmax),
                jnp.where(lane_is_s, sarg, segarg))

    b0 = wid * ROWS_PER_W
    b1 = b0 + 1

    a0 = pltpu.async_copy(att_hbm.at[b0], att0_v, s_a0)
    a1 = pltpu.async_copy(att_hbm.at[b1], att1_v, s_a1)
    a0.wait()
    a1.wait()

    def build(s, carry):
        sm0, sa0, sm1, sa1 = carry
        sm0, sa0 = build_one(att0_v, s, sm0, sa0)
        sm1, sa1 = build_one(att1_v, s, sm1, sa1)
        return (sm0, sa0, sm1, sa1)

    neg = jnp.full((NLANE,), _NEG_INF, jnp.float32)
    zer = jnp.zeros((NLANE,), jnp.int32)
    sm0, sa0, sm1, sa1 = lax.fori_loop(0, NSEG, build, (neg, zer, neg, zer))

    def extract_lo(k, carry):
        sm0, sa0, sm1, sa1, lo0, lo1 = carry
        g0_vec, sm0, sa0 = extract_one(att0_v, sm0, sa0)
        g1_vec, sm1, sa1 = extract_one(att1_v, sm1, sa1)
        lane_is_k = lanes == k
        lo0 = jnp.where(lane_is_k, b0 * S + g0_vec, lo0)
        lo1 = jnp.where(lane_is_k, b1 * S + g1_vec, lo1)
        return (sm0, sa0, sm1, sa1, lo0, lo1)

    sm0, sa0, sm1, sa1, lo0, lo1 = lax.fori_loop(
        0, NLANE, extract_lo, (sm0, sa0, sm1, sa1, zer, zer))

    g0a = pltpu.async_copy(ctx_hbm.at[lo0], rows0_v.at[pl.ds(0, NLANE)], s_g0)
    g1a = pltpu.async_copy(ctx_hbm.at[lo1], rows1_v.at[pl.ds(0, NLANE)], s_g1)

    def extract_hi(k, carry):
        sm0, sa0, sm1, sa1, hi0, hi1 = carry
        g0_vec, sm0, sa0 = extract_one(att0_v, sm0, sa0)
        g1_vec, sm1, sa1 = extract_one(att1_v, sm1, sa1)
        lane_is_k = lanes == k
        hi0 = jnp.where(lane_is_k, b0 * S + g0_vec, hi0)
        hi1 = jnp.where(lane_is_k, b1 * S + g1_vec, hi1)
        return (sm0, sa0, sm1, sa1, hi0, hi1)

    _, _, _, _, hi0, hi1 = lax.fori_loop(
        0, NLANE, extract_hi, (sm0, sa0, sm1, sa1, zer, zer))

    g0b = pltpu.async_copy(ctx_hbm.at[hi0], rows0_v.at[pl.ds(NLANE, NLANE)],
                           s_g0)
    g1b = pltpu.async_copy(ctx_hbm.at[hi1], rows1_v.at[pl.ds(NLANE, NLANE)],
                           s_g1)

    g0a.wait()
    g0b.wait()
    o0 = pltpu.async_copy(rows0_v, out_hbm.at[b0], s_o0)
    g1a.wait()
    g1b.wait()
    o1 = pltpu.async_copy(rows1_v, out_hbm.at[b1], s_o1)
    o0.wait()
    o1.wait()


_sc_call = pl.kernel(
    _tec_body,
    out_type=jax.ShapeDtypeStruct((B, KV, D), jnp.float32),
    mesh=plsc.VectorSubcoreMesh(core_axis_name="c", subcore_axis_name="s"),
    scratch_types=[
        pltpu.VMEM((S,), jnp.float32),
        pltpu.VMEM((S,), jnp.float32),
        pltpu.VMEM((KV, D), jnp.float32),
        pltpu.VMEM((KV, D), jnp.float32),
        pltpu.SemaphoreType.DMA,
        pltpu.SemaphoreType.DMA,
        pltpu.SemaphoreType.DMA,
        pltpu.SemaphoreType.DMA,
        pltpu.SemaphoreType.DMA,
        pltpu.SemaphoreType.DMA,
    ],
)


def kernel(attention, context, K):
    del K
    return _sc_call(attention, context.reshape(B * S, D))
